# baseline (speedup 1.0000x reference)
import functools

import jax
import jax.numpy as jnp
from jax.experimental import pallas as pl
from jax.experimental.pallas import tpu as pltpu
from jax.experimental.pallas import tpu_sc as plsc

_N = 10000
_E = 320000
_B = 50
_NPG = 200
_F = 128
_K1, _K2, _K3 = 160, 128, 103
_NP = 256
_GROW = 200
_GSZ = _GROW * _NP
_ABUF = 2 * _GSZ
_CH = 3200
_NCH = _E // _CH


def _adj_body(edge_hbm, zin_hbm, out_hbm, abuf, sbuf, dbuf, sem_s, sem_d):
    c = jax.lax.axis_index("c")
    s = jax.lax.axis_index("s")
    w = s * 2 + c
    g0 = w
    g1 = w + 32

    pltpu.sync_copy(zin_hbm, abuf)

    ones16 = jnp.ones((16,), jnp.float32)
    one_i = jnp.full((16,), 1, jnp.int32)
    g0v = jnp.full((16,), g0, jnp.int32)
    g1v = jnp.full((16,), g1, jnp.int32)
    npg_v = jnp.full((16,), _NPG, jnp.int32)
    np_v = jnp.full((16,), _NP, jnp.int32)
    gsz_v = jnp.full((16,), _GSZ, jnp.int32)
    magic_v = jnp.full((16,), 20972, jnp.int32)
    zero_v = jnp.zeros((16,), jnp.int32)
    shift_v = jnp.full((16,), 22, jnp.int32)

    def _start(ci, slot):
        off = pl.multiple_of(ci * _CH, 8)
        so = pl.multiple_of(slot * _CH, 8)
        pltpu.async_copy(edge_hbm.at[0, pl.ds(off, _CH)],
                         sbuf.at[pl.ds(so, _CH)], sem_s)
        pltpu.async_copy(edge_hbm.at[1, pl.ds(off, _CH)],
                         dbuf.at[pl.ds(so, _CH)], sem_d)

    def _wait(slot):
        so = pl.multiple_of(slot * _CH, 8)
        pltpu.make_async_copy(edge_hbm.at[0, pl.ds(0, _CH)],
                              sbuf.at[pl.ds(so, _CH)], sem_s).wait()
        pltpu.make_async_copy(edge_hbm.at[1, pl.ds(0, _CH)],
                              dbuf.at[pl.ds(so, _CH)], sem_d).wait()

    _start(0, 0)

    def chunk_body(ci, carry):
        slot = jax.lax.rem(ci, 2)
        _wait(slot)

        @pl.when(ci + 1 < _NCH)
        def _():
            _start(ci + 1, 1 - slot)

        @plsc.parallel_loop(0, _CH // 16, unroll=8)
        def _eloop(i):
            eo = pl.multiple_of(slot * _CH + i * 16, 16)
            sv = sbuf[pl.ds(eo, 16)]
            dv = dbuf[pl.ds(eo, 16)]
            g = jnp.right_shift(sv * magic_v, shift_v)
            sl = sv - g * npg_v
            dl = dv - g * npg_v
            is1 = g == g1v
            m = (g == g0v) | is1
            slot_v = jnp.where(is1, one_i, zero_v)
            plsc.addupdate_scatter(abuf, [slot_v, dl, sl], ones16, mask=m)

        return carry

    jax.lax.fori_loop(0, _NCH, chunk_body, 0)

    pltpu.sync_copy(abuf.at[0], out_hbm.at[g0])

    @pl.when(w < _B - 32)
    def _():
        pltpu.sync_copy(abuf.at[1], out_hbm.at[g1])


_ADJ_CACHE = []


def _adj_build(edge_index, zin):
    if not _ADJ_CACHE:
        _ADJ_CACHE.append(functools.partial(
            pl.kernel,
            mesh=plsc.VectorSubcoreMesh(core_axis_name="c",
                                        subcore_axis_name="s"),
            out_type=jax.ShapeDtypeStruct((_B, _GROW, _NP), jnp.float32),
            scratch_types=[
                pltpu.VMEM((2, _GROW, _NP), jnp.float32),
                pltpu.VMEM((2 * _CH,), jnp.int32),
                pltpu.VMEM((2 * _CH,), jnp.int32),
                pltpu.SemaphoreType.DMA,
                pltpu.SemaphoreType.DMA,
            ],
            compiler_params=pltpu.CompilerParams(needs_layout_passes=False),
        )(_adj_body))
    return _ADJ_CACHE[0](edge_index, zin)


_PREC = jax.lax.Precision.HIGHEST


def _dot(a, b):
    return jax.lax.dot_general(a, b, (((1,), (0,)), ((), ())),
                               precision=_PREC,
                               preferred_element_type=jnp.float32)


def _dot_nt(a, b):
    return jax.lax.dot_general(a, b, (((1,), (1,)), ((), ())),
                               precision=_PREC,
                               preferred_element_type=jnp.float32)


def _pipe_body(x_ref, a_ref, w1_ref, b1_ref, p1_ref, w2_ref, b2_ref, p2_ref,
               w3_ref, b3_ref, p3_ref, out_ref):
    h = jnp.concatenate(
        [x_ref[0], jnp.zeros((_NP - _NPG, _F), jnp.float32)], axis=0)
    A = jnp.concatenate(
        [a_ref[0], jnp.zeros((_NP - _GROW, _NP), jnp.float32)], axis=0)

    ltij = (jax.lax.broadcasted_iota(jnp.int32, (_NPG, _NPG), 0)
            < jax.lax.broadcasted_iota(jnp.int32, (_NPG, _NPG), 1))
    kc = (jax.lax.broadcasted_iota(jnp.int32, (_NP, 1), 0)
          < _NPG).astype(jnp.float32)
    ones_row = jnp.ones((1, _NPG), jnp.float32)
    pad_col = jnp.zeros((_NP - _NPG, 1), jnp.float32)

    layers = ((w1_ref, b1_ref, p1_ref, _K1),
              (w2_ref, b2_ref, p2_ref, _K2),
              (w3_ref, b3_ref, p3_ref, _K3))

    for w_ref, b_ref, p_ref, k in layers:
        W = w_ref[...]
        b = b_ref[...]
        p = p_ref[...]
        xw = _dot(h, W)
        deg = _dot(A, kc) + 1.0
        dinv = jax.lax.rsqrt(deg)
        dxw = dinv * xw
        z = dinv * (_dot(A, dxw) + dxw) + b
        hc = jnp.maximum(z, 0.0)

        pn = p / jnp.sqrt(jnp.sum(p * p))
        s_col = jnp.sum(hc * pn, axis=1, keepdims=True)
        sm_col = jnp.where(kc > 0, s_col, -jnp.inf)
        sm = sm_col[:_NPG]
        sm_row = jnp.transpose(sm)

        beats = ((sm > sm_row)
                 | ((sm == sm_row) & ltij)).astype(jnp.float32)
        rank = _dot(ones_row, beats)
        kept = (rank < k).astype(jnp.float32)
        kc = jnp.concatenate([jnp.transpose(kept), pad_col], axis=0)

        h = hc * jnp.tanh(s_col) * kc

    pooled = jnp.sum(h, axis=0, keepdims=True) * (1.0 / _K3)
    out_ref[...] = jnp.broadcast_to(pooled, (8, _F))[None]


def _head_body(x_ref, w1_ref, b1_ref, w2_ref, b2_ref, out_ref):
    z1 = jnp.maximum(_dot(x_ref[...], w1_ref[...]) + b1_ref[...], 0.0)
    z2 = _dot(z1, w2_ref[...]) + b2_ref[...]
    m = jnp.max(z2, axis=1, keepdims=True)
    e = jnp.exp(z2 - m)
    lse = jnp.log(jnp.sum(e, axis=1, keepdims=True))
    out_ref[...] = z2 - m - lse


def _full(shape):
    return pl.BlockSpec(shape, lambda *a: tuple(0 for _ in shape))


def kernel(x, edge_index, batch, W1, b1, p1, W2, b2, p2, W3, b3, p3,
           lw1, lb1, lw2, lb2):
    zin = jnp.zeros((2, _GROW, _NP), jnp.float32)
    A = _adj_build(edge_index.astype(jnp.int32), zin)
    xr = x.reshape(_B, _NPG, _F)

    pooled = pl.pallas_call(
        _pipe_body,
        grid=(_B,),
        in_specs=[
            pl.BlockSpec((1, _NPG, _F), lambda g: (g, 0, 0)),
            pl.BlockSpec((1, _GROW, _NP), lambda g: (g, 0, 0)),
            _full((_F, _F)), _full((1, _F)), _full((1, _F)),
            _full((_F, _F)), _full((1, _F)), _full((1, _F)),
            _full((_F, _F)), _full((1, _F)), _full((1, _F)),
        ],
        out_specs=pl.BlockSpec((1, 8, _F), lambda g: (g, 0, 0)),
        out_shape=jax.ShapeDtypeStruct((_B, 8, _F), jnp.float32),
    )(xr, A,
      W1, b1.reshape(1, _F), p1.reshape(1, _F),
      W2, b2.reshape(1, _F), p2.reshape(1, _F),
      W3, b3.reshape(1, _F), p3.reshape(1, _F))
    pooled = pooled[:, 0, :]

    pooled_pad = jnp.zeros((64, _F), jnp.float32).at[:_B].set(pooled)
    lw1p = jnp.zeros((_F, _F), jnp.float32).at[:, :64].set(lw1)
    lb1p = jnp.zeros((1, _F), jnp.float32).at[0, :64].set(lb1)
    lw2p = jnp.zeros((_F, _F), jnp.float32).at[:64, :10].set(lw2)
    lb2p = jnp.full((1, _F), -1e30, jnp.float32).at[0, :10].set(lb2)

    out = pl.pallas_call(
        _head_body,
        in_specs=[_full((64, _F)), _full((_F, _F)), _full((1, _F)),
                  _full((_F, _F)), _full((1, _F))],
        out_specs=_full((64, _F)),
        out_shape=jax.ShapeDtypeStruct((64, _F), jnp.float32),
    )(pooled_pad, lw1p, lb1p, lw2p, lb2p)

    return out[:_B, :10]


# speedup vs baseline: 1.4278x; 1.4278x over previous
import functools

import jax
import jax.numpy as jnp
from jax.experimental import pallas as pl
from jax.experimental.pallas import tpu as pltpu
from jax.experimental.pallas import tpu_sc as plsc

_N = 10000
_E = 320000
_B = 50
_NPG = 200
_F = 128
_K1, _K2, _K3 = 160, 128, 103
_NP = 256
_GROW = 200
_GSZ = _GROW * _NP
_ABUF = 2 * _GSZ
_CH = 3200
_NCH = _E // _CH


def _adj_body(edge_hbm, zin_hbm, out_hbm, abuf, sbuf, dbuf, sem_s, sem_d):
    c = jax.lax.axis_index("c")
    s = jax.lax.axis_index("s")
    w = s * 2 + c
    g0 = w
    g1 = w + 32

    pltpu.sync_copy(zin_hbm, abuf)

    ones16 = jnp.ones((16,), jnp.float32)
    one_i = jnp.full((16,), 1, jnp.int32)
    g0v = jnp.full((16,), g0, jnp.int32)
    g1v = jnp.full((16,), g1, jnp.int32)
    npg_v = jnp.full((16,), _NPG, jnp.int32)
    np_v = jnp.full((16,), _NP, jnp.int32)
    gsz_v = jnp.full((16,), _GSZ, jnp.int32)
    magic_v = jnp.full((16,), 20972, jnp.int32)
    zero_v = jnp.zeros((16,), jnp.int32)
    shift_v = jnp.full((16,), 22, jnp.int32)

    def _start(ci, slot):
        off = pl.multiple_of(ci * _CH, 8)
        so = pl.multiple_of(slot * _CH, 8)
        pltpu.async_copy(edge_hbm.at[0, pl.ds(off, _CH)],
                         sbuf.at[pl.ds(so, _CH)], sem_s)
        pltpu.async_copy(edge_hbm.at[1, pl.ds(off, _CH)],
                         dbuf.at[pl.ds(so, _CH)], sem_d)

    def _wait(slot):
        so = pl.multiple_of(slot * _CH, 8)
        pltpu.make_async_copy(edge_hbm.at[0, pl.ds(0, _CH)],
                              sbuf.at[pl.ds(so, _CH)], sem_s).wait()
        pltpu.make_async_copy(edge_hbm.at[1, pl.ds(0, _CH)],
                              dbuf.at[pl.ds(so, _CH)], sem_d).wait()

    _start(0, 0)

    def chunk_body(ci, carry):
        slot = jax.lax.rem(ci, 2)
        _wait(slot)

        @pl.when(ci + 1 < _NCH)
        def _():
            _start(ci + 1, 1 - slot)

        @plsc.parallel_loop(0, _CH // 16, unroll=8)
        def _eloop(i):
            eo = pl.multiple_of(slot * _CH + i * 16, 16)
            sv = sbuf[pl.ds(eo, 16)]
            dv = dbuf[pl.ds(eo, 16)]
            g = jnp.right_shift(sv * magic_v, shift_v)
            sl = sv - g * npg_v
            dl = dv - g * npg_v
            is1 = g == g1v
            m = (g == g0v) | is1
            slot_v = jnp.where(is1, one_i, zero_v)
            plsc.addupdate_scatter(abuf, [slot_v, dl, sl], ones16, mask=m)

        return carry

    jax.lax.fori_loop(0, _NCH, chunk_body, 0)

    pltpu.sync_copy(abuf.at[0], out_hbm.at[g0])

    @pl.when(w < _B - 32)
    def _():
        pltpu.sync_copy(abuf.at[1], out_hbm.at[g1])


_ADJ_CACHE = []


def _adj_build(edge_index, zin):
    if not _ADJ_CACHE:
        _ADJ_CACHE.append(functools.partial(
            pl.kernel,
            mesh=plsc.VectorSubcoreMesh(core_axis_name="c",
                                        subcore_axis_name="s"),
            out_type=jax.ShapeDtypeStruct((_B, _GROW, _NP), jnp.float32),
            scratch_types=[
                pltpu.VMEM((2, _GROW, _NP), jnp.float32),
                pltpu.VMEM((2 * _CH,), jnp.int32),
                pltpu.VMEM((2 * _CH,), jnp.int32),
                pltpu.SemaphoreType.DMA,
                pltpu.SemaphoreType.DMA,
            ],
            compiler_params=pltpu.CompilerParams(needs_layout_passes=False),
        )(_adj_body))
    return _ADJ_CACHE[0](edge_index, zin)


_PREC = jax.lax.Precision.HIGHEST


def _dot(a, b):
    return jax.lax.dot_general(a, b, (((1,), (0,)), ((), ())),
                               precision=_PREC,
                               preferred_element_type=jnp.float32)


def _dot_nt(a, b):
    return jax.lax.dot_general(a, b, (((1,), (1,)), ((), ())),
                               precision=_PREC,
                               preferred_element_type=jnp.float32)


def _graph_pipe(x, A0, wbp):
    h = jnp.concatenate(
        [x, jnp.zeros((_NP - _NPG, _F), jnp.float32)], axis=0)
    A = jnp.concatenate(
        [A0, jnp.zeros((_NP - _GROW, _NP), jnp.float32)], axis=0)

    ii = jax.lax.broadcasted_iota(jnp.int32, (_NP, _NP), 0)
    ij = jax.lax.broadcasted_iota(jnp.int32, (_NP, _NP), 1)

    kc = (jax.lax.broadcasted_iota(jnp.int32, (_NP, 1), 0)
          < _NPG).astype(jnp.float32)
    kr = jnp.transpose(kc)

    for W, b, p, k in wbp:
        xw = _dot(h, W)
        deg = jnp.sum(A, axis=1, keepdims=True) + 1.0
        dinv = jax.lax.rsqrt(deg)
        dxw = dinv * xw
        z = dinv * (_dot(A, dxw) + dxw) + b
        hc = jnp.maximum(z, 0.0)

        pn = p / jnp.sqrt(jnp.sum(p * p))
        s_col = jnp.sum(hc * pn, axis=1, keepdims=True)
        sm_col = jnp.where(kc > 0, s_col, -jnp.inf)
        sm_row = jnp.transpose(sm_col)

        beats = (sm_col > sm_row) | ((sm_col == sm_row) & (ii < ij))
        rank_row = jnp.sum(beats.astype(jnp.int32), axis=0, keepdims=True)
        kr = kr * (rank_row < k).astype(jnp.float32)
        kc = jnp.transpose(kr)

        h = hc * jnp.tanh(s_col) * kc
        A = A * kc * kr

    return jnp.sum(h, axis=0, keepdims=True) * (1.0 / _K3)


_GPB = 2


def _pipe_body(x_ref, a_ref, w1_ref, b1_ref, p1_ref, w2_ref, b2_ref, p2_ref,
               w3_ref, b3_ref, p3_ref, out_ref):
    wbp = ((w1_ref[...], b1_ref[...], p1_ref[...], _K1),
           (w2_ref[...], b2_ref[...], p2_ref[...], _K2),
           (w3_ref[...], b3_ref[...], p3_ref[...], _K3))
    for gi in range(_GPB):
        pooled = _graph_pipe(x_ref[gi], a_ref[gi], wbp)
        out_ref[gi] = jnp.broadcast_to(pooled, (8, _F))


def _head_body(x_ref, w1_ref, b1_ref, w2_ref, b2_ref, out_ref):
    z1 = jnp.maximum(_dot(x_ref[...], w1_ref[...]) + b1_ref[...], 0.0)
    z2 = _dot(z1, w2_ref[...]) + b2_ref[...]
    m = jnp.max(z2, axis=1, keepdims=True)
    e = jnp.exp(z2 - m)
    lse = jnp.log(jnp.sum(e, axis=1, keepdims=True))
    out_ref[...] = z2 - m - lse


def _full(shape):
    return pl.BlockSpec(shape, lambda *a: tuple(0 for _ in shape))


def kernel(x, edge_index, batch, W1, b1, p1, W2, b2, p2, W3, b3, p3,
           lw1, lb1, lw2, lb2):
    zin = jnp.zeros((2, _GROW, _NP), jnp.float32)
    A = _adj_build(edge_index.astype(jnp.int32), zin)
    xr = x.reshape(_B, _NPG, _F)

    pooled = pl.pallas_call(
        _pipe_body,
        grid=(_B // _GPB,),
        in_specs=[
            pl.BlockSpec((_GPB, _NPG, _F), lambda g: (g, 0, 0)),
            pl.BlockSpec((_GPB, _GROW, _NP), lambda g: (g, 0, 0)),
            _full((_F, _F)), _full((1, _F)), _full((1, _F)),
            _full((_F, _F)), _full((1, _F)), _full((1, _F)),
            _full((_F, _F)), _full((1, _F)), _full((1, _F)),
        ],
        out_specs=pl.BlockSpec((_GPB, 8, _F), lambda g: (g, 0, 0)),
        out_shape=jax.ShapeDtypeStruct((_B, 8, _F), jnp.float32),
    )(xr, A,
      W1, b1.reshape(1, _F), p1.reshape(1, _F),
      W2, b2.reshape(1, _F), p2.reshape(1, _F),
      W3, b3.reshape(1, _F), p3.reshape(1, _F))
    pooled = pooled[:, 0, :]

    pooled_pad = jnp.zeros((64, _F), jnp.float32).at[:_B].set(pooled)
    lw1p = jnp.zeros((_F, _F), jnp.float32).at[:, :64].set(lw1)
    lb1p = jnp.zeros((1, _F), jnp.float32).at[0, :64].set(lb1)
    lw2p = jnp.zeros((_F, _F), jnp.float32).at[:64, :10].set(lw2)
    lb2p = jnp.full((1, _F), -1e30, jnp.float32).at[0, :10].set(lb2)

    out = pl.pallas_call(
        _head_body,
        in_specs=[_full((64, _F)), _full((_F, _F)), _full((1, _F)),
                  _full((_F, _F)), _full((1, _F))],
        out_specs=_full((64, _F)),
        out_shape=jax.ShapeDtypeStruct((64, _F), jnp.float32),
    )(pooled_pad, lw1p, lb1p, lw2p, lb2p)

    return out[:_B, :10]


# speedup vs baseline: 1.5112x; 1.0584x over previous
import functools

import jax
import jax.numpy as jnp
from jax.experimental import pallas as pl
from jax.experimental.pallas import tpu as pltpu
from jax.experimental.pallas import tpu_sc as plsc

_N = 10000
_E = 320000
_B = 50
_NPG = 200
_F = 128
_K1, _K2, _K3 = 160, 128, 103
_NP = 256
_GROW = 200
_GSZ = _GROW * _NP
_ABUF = 2 * _GSZ
_CH = 3200
_NCH = _E // _CH


def _adj_body(edge_hbm, zin_hbm, out_hbm, abuf, sbuf, dbuf, sem_s, sem_d):
    c = jax.lax.axis_index("c")
    s = jax.lax.axis_index("s")
    w = s * 2 + c
    g0 = w
    g1 = w + 32

    pltpu.sync_copy(zin_hbm, abuf)

    ones16 = jnp.ones((16,), jnp.float32)
    one_i = jnp.full((16,), 1, jnp.int32)
    g0v = jnp.full((16,), g0, jnp.int32)
    g1v = jnp.full((16,), g1, jnp.int32)
    npg_v = jnp.full((16,), _NPG, jnp.int32)
    np_v = jnp.full((16,), _NP, jnp.int32)
    gsz_v = jnp.full((16,), _GSZ, jnp.int32)
    magic_v = jnp.full((16,), 20972, jnp.int32)
    zero_v = jnp.zeros((16,), jnp.int32)
    shift_v = jnp.full((16,), 22, jnp.int32)

    def _start(ci, slot):
        off = pl.multiple_of(ci * _CH, 8)
        so = pl.multiple_of(slot * _CH, 8)
        pltpu.async_copy(edge_hbm.at[0, pl.ds(off, _CH)],
                         sbuf.at[pl.ds(so, _CH)], sem_s)
        pltpu.async_copy(edge_hbm.at[1, pl.ds(off, _CH)],
                         dbuf.at[pl.ds(so, _CH)], sem_d)

    def _wait(slot):
        so = pl.multiple_of(slot * _CH, 8)
        pltpu.make_async_copy(edge_hbm.at[0, pl.ds(0, _CH)],
                              sbuf.at[pl.ds(so, _CH)], sem_s).wait()
        pltpu.make_async_copy(edge_hbm.at[1, pl.ds(0, _CH)],
                              dbuf.at[pl.ds(so, _CH)], sem_d).wait()

    _start(0, 0)

    def chunk_body(ci, carry):
        slot = jax.lax.rem(ci, 2)
        _wait(slot)

        @pl.when(ci + 1 < _NCH)
        def _():
            _start(ci + 1, 1 - slot)

        @plsc.parallel_loop(0, _CH // 16, unroll=8)
        def _eloop(i):
            eo = pl.multiple_of(slot * _CH + i * 16, 16)
            sv = sbuf[pl.ds(eo, 16)]
            dv = dbuf[pl.ds(eo, 16)]
            g = jnp.right_shift(sv * magic_v, shift_v)
            sl = sv - g * npg_v
            dl = dv - g * npg_v
            is1 = g == g1v
            m = (g == g0v) | is1
            slot_v = jnp.where(is1, one_i, zero_v)
            plsc.addupdate_scatter(abuf, [slot_v, dl, sl], ones16, mask=m)

        return carry

    jax.lax.fori_loop(0, _NCH, chunk_body, 0)

    pltpu.sync_copy(abuf.at[0], out_hbm.at[g0])

    @pl.when(w < _B - 32)
    def _():
        pltpu.sync_copy(abuf.at[1], out_hbm.at[g1])


_ADJ_CACHE = []


def _adj_build(edge_index, zin):
    if not _ADJ_CACHE:
        _ADJ_CACHE.append(functools.partial(
            pl.kernel,
            mesh=plsc.VectorSubcoreMesh(core_axis_name="c",
                                        subcore_axis_name="s"),
            out_type=jax.ShapeDtypeStruct((_B, _GROW, _NP), jnp.float32),
            scratch_types=[
                pltpu.VMEM((2, _GROW, _NP), jnp.float32),
                pltpu.VMEM((2 * _CH,), jnp.int32),
                pltpu.VMEM((2 * _CH,), jnp.int32),
                pltpu.SemaphoreType.DMA,
                pltpu.SemaphoreType.DMA,
            ],
            compiler_params=pltpu.CompilerParams(needs_layout_passes=False),
        )(_adj_body))
    return _ADJ_CACHE[0](edge_index, zin)


_PREC = jax.lax.Precision.HIGHEST


def _dot(a, b):
    return jax.lax.dot_general(a, b, (((1,), (0,)), ((), ())),
                               precision=_PREC,
                               preferred_element_type=jnp.float32)


def _dot_nt(a, b):
    return jax.lax.dot_general(a, b, (((1,), (1,)), ((), ())),
                               precision=_PREC,
                               preferred_element_type=jnp.float32)


def _graph_pipe(x, A0, wbp):
    h = jnp.concatenate(
        [x, jnp.zeros((_NP - _NPG, _F), jnp.float32)], axis=0)
    A = jnp.concatenate(
        [A0, jnp.zeros((_NP - _GROW, _NP), jnp.float32)], axis=0)

    ii = jax.lax.broadcasted_iota(jnp.int32, (_NP, _NP), 0)
    ij = jax.lax.broadcasted_iota(jnp.int32, (_NP, _NP), 1)

    kc = (jax.lax.broadcasted_iota(jnp.int32, (_NP, 1), 0)
          < _NPG).astype(jnp.float32)
    kr = jnp.transpose(kc)

    for W, b, p, k in wbp:
        xw = _dot(h, W)
        deg = jnp.sum(A, axis=1, keepdims=True) + 1.0
        dinv = jax.lax.rsqrt(deg)
        dxw = dinv * xw
        z = dinv * (_dot(A, dxw) + dxw) + b
        hc = jnp.maximum(z, 0.0)

        pn = p / jnp.sqrt(jnp.sum(p * p))
        s_col = jnp.sum(hc * pn, axis=1, keepdims=True)
        sm_col = jnp.where(kc > 0, s_col, -jnp.inf)
        sm_row = jnp.transpose(sm_col)

        beats = (sm_col > sm_row) | ((sm_col == sm_row) & (ii < ij))
        rank_row = jnp.sum(beats.astype(jnp.int32), axis=0, keepdims=True)
        kr = kr * (rank_row < k).astype(jnp.float32)
        kc = jnp.transpose(kr)

        h = hc * jnp.tanh(s_col) * kc
        A = A * kc * kr

    return jnp.sum(h, axis=0, keepdims=True) * (1.0 / _K3)


_GPB = 5


def _pipe_body(x_ref, a_ref, w1_ref, b1_ref, p1_ref, w2_ref, b2_ref, p2_ref,
               w3_ref, b3_ref, p3_ref, out_ref):
    wbp = ((w1_ref[...], b1_ref[...], p1_ref[...], _K1),
           (w2_ref[...], b2_ref[...], p2_ref[...], _K2),
           (w3_ref[...], b3_ref[...], p3_ref[...], _K3))
    for gi in range(_GPB):
        pooled = _graph_pipe(x_ref[gi], a_ref[gi], wbp)
        out_ref[gi] = jnp.broadcast_to(pooled, (8, _F))


def _head_body(x_ref, w1_ref, b1_ref, w2_ref, b2_ref, out_ref):
    z1 = jnp.maximum(_dot(x_ref[...], w1_ref[...]) + b1_ref[...], 0.0)
    z2 = _dot(z1, w2_ref[...]) + b2_ref[...]
    m = jnp.max(z2, axis=1, keepdims=True)
    e = jnp.exp(z2 - m)
    lse = jnp.log(jnp.sum(e, axis=1, keepdims=True))
    out_ref[...] = z2 - m - lse


def _full(shape):
    return pl.BlockSpec(shape, lambda *a: tuple(0 for _ in shape))


def kernel(x, edge_index, batch, W1, b1, p1, W2, b2, p2, W3, b3, p3,
           lw1, lb1, lw2, lb2):
    zin = jnp.zeros((2, _GROW, _NP), jnp.float32)
    A = _adj_build(edge_index.astype(jnp.int32), zin)
    xr = x.reshape(_B, _NPG, _F)

    pooled = pl.pallas_call(
        _pipe_body,
        grid=(_B // _GPB,),
        in_specs=[
            pl.BlockSpec((_GPB, _NPG, _F), lambda g: (g, 0, 0)),
            pl.BlockSpec((_GPB, _GROW, _NP), lambda g: (g, 0, 0)),
            _full((_F, _F)), _full((1, _F)), _full((1, _F)),
            _full((_F, _F)), _full((1, _F)), _full((1, _F)),
            _full((_F, _F)), _full((1, _F)), _full((1, _F)),
        ],
        out_specs=pl.BlockSpec((_GPB, 8, _F), lambda g: (g, 0, 0)),
        out_shape=jax.ShapeDtypeStruct((_B, 8, _F), jnp.float32),
    )(xr, A,
      W1, b1.reshape(1, _F), p1.reshape(1, _F),
      W2, b2.reshape(1, _F), p2.reshape(1, _F),
      W3, b3.reshape(1, _F), p3.reshape(1, _F))
    pooled = pooled[:, 0, :]

    pooled_pad = jnp.zeros((64, _F), jnp.float32).at[:_B].set(pooled)
    lw1p = jnp.zeros((_F, _F), jnp.float32).at[:, :64].set(lw1)
    lb1p = jnp.zeros((1, _F), jnp.float32).at[0, :64].set(lb1)
    lw2p = jnp.zeros((_F, _F), jnp.float32).at[:64, :10].set(lw2)
    lb2p = jnp.full((1, _F), -1e30, jnp.float32).at[0, :10].set(lb2)

    out = pl.pallas_call(
        _head_body,
        in_specs=[_full((64, _F)), _full((_F, _F)), _full((1, _F)),
                  _full((_F, _F)), _full((1, _F))],
        out_specs=_full((64, _F)),
        out_shape=jax.ShapeDtypeStruct((64, _F), jnp.float32),
    )(pooled_pad, lw1p, lb1p, lw2p, lb2p)

    return out[:_B, :10]


# speedup vs baseline: 1.5551x; 1.0290x over previous
import functools

import jax
import jax.numpy as jnp
from jax.experimental import pallas as pl
from jax.experimental.pallas import tpu as pltpu
from jax.experimental.pallas import tpu_sc as plsc

_N = 10000
_E = 320000
_B = 50
_NPG = 200
_F = 128
_K1, _K2, _K3 = 160, 128, 103
_NP = 256
_GROW = 200
_GSZ = _GROW * _NP
_ABUF = 2 * _GSZ
_CH = 3200
_NCH = _E // _CH


def _adj_body(edge_hbm, zin_hbm, out_hbm, abuf, sbuf, dbuf, sem_s, sem_d):
    c = jax.lax.axis_index("c")
    s = jax.lax.axis_index("s")
    w = s * 2 + c
    g0 = w
    g1 = w + 32

    pltpu.sync_copy(zin_hbm, abuf)

    ones16 = jnp.ones((16,), jnp.float32)
    one_i = jnp.full((16,), 1, jnp.int32)
    g0v = jnp.full((16,), g0, jnp.int32)
    g1v = jnp.full((16,), g1, jnp.int32)
    npg_v = jnp.full((16,), _NPG, jnp.int32)
    np_v = jnp.full((16,), _NP, jnp.int32)
    gsz_v = jnp.full((16,), _GSZ, jnp.int32)
    magic_v = jnp.full((16,), 20972, jnp.int32)
    zero_v = jnp.zeros((16,), jnp.int32)
    shift_v = jnp.full((16,), 22, jnp.int32)

    def _start(ci, slot):
        off = pl.multiple_of(ci * _CH, 8)
        so = pl.multiple_of(slot * _CH, 8)
        pltpu.async_copy(edge_hbm.at[0, pl.ds(off, _CH)],
                         sbuf.at[pl.ds(so, _CH)], sem_s)
        pltpu.async_copy(edge_hbm.at[1, pl.ds(off, _CH)],
                         dbuf.at[pl.ds(so, _CH)], sem_d)

    def _wait(slot):
        so = pl.multiple_of(slot * _CH, 8)
        pltpu.make_async_copy(edge_hbm.at[0, pl.ds(0, _CH)],
                              sbuf.at[pl.ds(so, _CH)], sem_s).wait()
        pltpu.make_async_copy(edge_hbm.at[1, pl.ds(0, _CH)],
                              dbuf.at[pl.ds(so, _CH)], sem_d).wait()

    _start(0, 0)

    def chunk_body(ci, carry):
        slot = jax.lax.rem(ci, 2)
        _wait(slot)

        @pl.when(ci + 1 < _NCH)
        def _():
            _start(ci + 1, 1 - slot)

        @plsc.parallel_loop(0, _CH // 16, unroll=8)
        def _eloop(i):
            eo = pl.multiple_of(slot * _CH + i * 16, 16)
            sv = sbuf[pl.ds(eo, 16)]
            dv = dbuf[pl.ds(eo, 16)]
            g = jnp.right_shift(sv * magic_v, shift_v)
            sl = sv - g * npg_v
            dl = dv - g * npg_v
            is1 = g == g1v
            m = (g == g0v) | is1
            slot_v = jnp.where(is1, one_i, zero_v)
            plsc.addupdate_scatter(abuf, [slot_v, dl, sl], ones16, mask=m)

        return carry

    jax.lax.fori_loop(0, _NCH, chunk_body, 0)

    pltpu.sync_copy(abuf.at[0], out_hbm.at[g0])

    @pl.when(w < _B - 32)
    def _():
        pltpu.sync_copy(abuf.at[1], out_hbm.at[g1])


_ADJ_CACHE = []


def _adj_build(edge_index, zin):
    if not _ADJ_CACHE:
        _ADJ_CACHE.append(functools.partial(
            pl.kernel,
            mesh=plsc.VectorSubcoreMesh(core_axis_name="c",
                                        subcore_axis_name="s"),
            out_type=jax.ShapeDtypeStruct((_B, _GROW, _NP), jnp.float32),
            scratch_types=[
                pltpu.VMEM((2, _GROW, _NP), jnp.float32),
                pltpu.VMEM((2 * _CH,), jnp.int32),
                pltpu.VMEM((2 * _CH,), jnp.int32),
                pltpu.SemaphoreType.DMA,
                pltpu.SemaphoreType.DMA,
            ],
            compiler_params=pltpu.CompilerParams(needs_layout_passes=False),
        )(_adj_body))
    return _ADJ_CACHE[0](edge_index, zin)


_PREC = jax.lax.Precision.HIGHEST


def _dot(a, b):
    return jax.lax.dot_general(a, b, (((1,), (0,)), ((), ())),
                               precision=_PREC,
                               preferred_element_type=jnp.float32)


def _dot_nt(a, b):
    return jax.lax.dot_general(a, b, (((1,), (1,)), ((), ())),
                               precision=_PREC,
                               preferred_element_type=jnp.float32)


def _graph_pipe(x, A0, wbp):
    h = jnp.concatenate(
        [x, jnp.zeros((_NP - _NPG, _F), jnp.float32)], axis=0)
    A = jnp.concatenate(
        [A0, jnp.zeros((_NP - _GROW, _NP), jnp.float32)], axis=0)

    ii = jax.lax.broadcasted_iota(jnp.int32, (_NP, _NP), 0)
    ij = jax.lax.broadcasted_iota(jnp.int32, (_NP, _NP), 1)

    kc = (jax.lax.broadcasted_iota(jnp.int32, (_NP, 1), 0)
          < _NPG).astype(jnp.float32)
    kr = jnp.transpose(kc)

    for W, b, p, k in wbp:
        xw = _dot(h, W)
        deg = jnp.sum(A, axis=1, keepdims=True) + 1.0
        dinv = jax.lax.rsqrt(deg)
        dxw = dinv * xw
        z = dinv * (_dot(A, dxw) + dxw) + b
        hc = jnp.maximum(z, 0.0)

        pn = p / jnp.sqrt(jnp.sum(p * p))
        s_col = jnp.sum(hc * pn, axis=1, keepdims=True)
        sm_col = jnp.where(kc > 0, s_col, -jnp.inf)
        sm_row = jnp.transpose(sm_col)

        beats = (sm_col > sm_row) | ((sm_col == sm_row) & (ii < ij))
        rank_row = jnp.sum(beats.astype(jnp.int32), axis=0, keepdims=True)
        kr = kr * (rank_row < k).astype(jnp.float32)
        kc = jnp.transpose(kr)

        h = hc * jnp.tanh(s_col) * kc
        A = A * kc * kr

    return jnp.sum(h, axis=0, keepdims=True) * (1.0 / _K3)


_GPB = 10


def _pipe_body(x_ref, a_ref, w1_ref, b1_ref, p1_ref, w2_ref, b2_ref, p2_ref,
               w3_ref, b3_ref, p3_ref, out_ref):
    wbp = ((w1_ref[...], b1_ref[...], p1_ref[...], _K1),
           (w2_ref[...], b2_ref[...], p2_ref[...], _K2),
           (w3_ref[...], b3_ref[...], p3_ref[...], _K3))
    for gi in range(_GPB):
        pooled = _graph_pipe(x_ref[gi], a_ref[gi], wbp)
        out_ref[gi] = jnp.broadcast_to(pooled, (8, _F))


def _head_body(x_ref, w1_ref, b1_ref, w2_ref, b2_ref, out_ref):
    z1 = jnp.maximum(_dot(x_ref[...], w1_ref[...]) + b1_ref[...], 0.0)
    z2 = _dot(z1, w2_ref[...]) + b2_ref[...]
    m = jnp.max(z2, axis=1, keepdims=True)
    e = jnp.exp(z2 - m)
    lse = jnp.log(jnp.sum(e, axis=1, keepdims=True))
    out_ref[...] = z2 - m - lse


def _full(shape):
    return pl.BlockSpec(shape, lambda *a: tuple(0 for _ in shape))


def kernel(x, edge_index, batch, W1, b1, p1, W2, b2, p2, W3, b3, p3,
           lw1, lb1, lw2, lb2):
    zin = jnp.zeros((2, _GROW, _NP), jnp.float32)
    A = _adj_build(edge_index.astype(jnp.int32), zin)
    xr = x.reshape(_B, _NPG, _F)

    pooled = pl.pallas_call(
        _pipe_body,
        grid=(_B // _GPB,),
        in_specs=[
            pl.BlockSpec((_GPB, _NPG, _F), lambda g: (g, 0, 0)),
            pl.BlockSpec((_GPB, _GROW, _NP), lambda g: (g, 0, 0)),
            _full((_F, _F)), _full((1, _F)), _full((1, _F)),
            _full((_F, _F)), _full((1, _F)), _full((1, _F)),
            _full((_F, _F)), _full((1, _F)), _full((1, _F)),
        ],
        out_specs=pl.BlockSpec((_GPB, 8, _F), lambda g: (g, 0, 0)),
        out_shape=jax.ShapeDtypeStruct((_B, 8, _F), jnp.float32),
    )(xr, A,
      W1, b1.reshape(1, _F), p1.reshape(1, _F),
      W2, b2.reshape(1, _F), p2.reshape(1, _F),
      W3, b3.reshape(1, _F), p3.reshape(1, _F))
    pooled = pooled[:, 0, :]

    pooled_pad = jnp.zeros((64, _F), jnp.float32).at[:_B].set(pooled)
    lw1p = jnp.zeros((_F, _F), jnp.float32).at[:, :64].set(lw1)
    lb1p = jnp.zeros((1, _F), jnp.float32).at[0, :64].set(lb1)
    lw2p = jnp.zeros((_F, _F), jnp.float32).at[:64, :10].set(lw2)
    lb2p = jnp.full((1, _F), -1e30, jnp.float32).at[0, :10].set(lb2)

    out = pl.pallas_call(
        _head_body,
        in_specs=[_full((64, _F)), _full((_F, _F)), _full((1, _F)),
                  _full((_F, _F)), _full((1, _F))],
        out_specs=_full((64, _F)),
        out_shape=jax.ShapeDtypeStruct((64, _F), jnp.float32),
    )(pooled_pad, lw1p, lb1p, lw2p, lb2p)

    return out[:_B, :10]
